# Initial kernel scaffold; baseline (speedup 1.0000x reference)
import jax
import jax.numpy as jnp
from jax.experimental import pallas as pl


def kernel(x, Wr, br, We, be, Wo, bo, norm_w):
    raise NotImplementedError("write your pallas kernel here")


# speedup vs baseline: 2.7704x; 2.7704x over previous
import functools

import jax
import jax.numpy as jnp
from jax.experimental import pallas as pl

_EPS = 1e-6


def _fused_body(x_ref, wr_ref, br_ref, we_ref, be_ref, wo_ref, bo_ref,
                nw_ref, o_ref):
    T, D = x_ref.shape
    E = wr_ref.shape[0]
    xf = x_ref[:]
    xb = xf.astype(jnp.bfloat16)

    logits = jax.lax.dot_general(
        xb, wr_ref[:], (((1,), (1,)), ((), ())),
        preferred_element_type=jnp.float32) + br_ref[:]

    idx = jax.lax.broadcasted_iota(jnp.int32, (T, E), 1)
    v1 = jnp.max(logits, axis=1, keepdims=True)
    i1 = jnp.min(jnp.where(logits == v1, idx, E), axis=1, keepdims=True)
    masked = jnp.where(idx == i1, -jnp.inf, logits)
    v2 = jnp.max(masked, axis=1, keepdims=True)
    i2 = jnp.min(jnp.where(masked == v2, idx, E), axis=1, keepdims=True)
    s = jnp.exp(v2 - v1)
    w1 = 1.0 / (1.0 + s)
    w2 = s * w1
    gates = jnp.where(idx == i1, w1, 0.0) + jnp.where(idx == i2, w2, 0.0)

    acc = jnp.zeros((T, D), jnp.float32)
    for e in range(E):
        h = jax.lax.dot_general(
            xb, we_ref[e], (((1,), (1,)), ((), ())),
            preferred_element_type=jnp.float32)
        h = h + be_ref[e:e + 1, :]
        h = h * jax.nn.sigmoid(h)
        acc = acc + gates[:, e:e + 1] * h

    ob = jax.lax.dot_general(
        acc.astype(jnp.bfloat16), wo_ref[:], (((1,), (1,)), ((), ())),
        preferred_element_type=jnp.float32) + bo_ref[:]
    y = xf + ob
    r = jax.lax.rsqrt(jnp.mean(y * y, axis=1, keepdims=True) + _EPS)
    o_ref[:] = (nw_ref[:] * y) * r


@jax.jit
def kernel(x, Wr, br, We, be, Wo, bo, norm_w):
    B, S, D = x.shape
    E = Wr.shape[0]
    N = B * S
    T = 256

    xf = x.reshape(N, D)
    Wr16 = Wr.astype(jnp.bfloat16)
    We16 = We.astype(jnp.bfloat16)
    Wo16 = Wo.astype(jnp.bfloat16)
    br2 = br.reshape(1, E)
    bo2 = bo.reshape(1, D)
    nw2 = norm_w.reshape(1, D)

    out = pl.pallas_call(
        _fused_body,
        grid=(N // T,),
        in_specs=[
            pl.BlockSpec((T, D), lambda i: (i, 0)),
            pl.BlockSpec((E, D), lambda i: (0, 0)),
            pl.BlockSpec((1, E), lambda i: (0, 0)),
            pl.BlockSpec((E, D, D), lambda i: (0, 0, 0)),
            pl.BlockSpec((E, D), lambda i: (0, 0)),
            pl.BlockSpec((D, D), lambda i: (0, 0)),
            pl.BlockSpec((1, D), lambda i: (0, 0)),
            pl.BlockSpec((1, D), lambda i: (0, 0)),
        ],
        out_specs=pl.BlockSpec((T, D), lambda i: (i, 0)),
        out_shape=jax.ShapeDtypeStruct((N, D), jnp.float32),
    )(xf, Wr16, br2, We16, be, Wo16, bo2, nw2)
    return out.reshape(B, S, D)
